# Initial kernel scaffold; baseline (speedup 1.0000x reference)
import jax
import jax.numpy as jnp
from jax.experimental import pallas as pl


def kernel(events, w1, b1, w2, b2, w3, b3):
    raise NotImplementedError("write your pallas kernel here")


# speedup vs baseline: 21.3472x; 21.3472x over previous
import functools

import jax
import jax.numpy as jnp
from jax import lax
from jax.experimental import pallas as pl
from jax.experimental.pallas import tpu as pltpu
from jax.experimental.pallas import tpu_sc as plsc

C = 9
H = 180
W = 240
B = 4
N = 1000000
HW = H * W
GHW = 8 * HW
N_PAD = 1 << 20
NB0 = 5
NB1 = 4

_A_CHUNK = 131072
_B_CHUNK = 16384
_SC_EV = N_PAD // 16
_SC_CHUNK = 8192
_ZB = 10800


def _leaky(v):
    return jnp.where(v >= 0, v, 0.1 * v)


def _tmax_kernel(t_ref, b_ref, o_ref):
    pid = pl.program_id(0)

    @pl.when(pid == 0)
    def _():
        o_ref[...] = jnp.zeros_like(o_ref)

    t = t_ref[...]
    b = b_ref[...]
    for k in range(B):
        mk = jnp.max(jnp.where(b == float(k), t, 0.0), axis=0)
        o_ref[k, :] = jnp.maximum(o_ref[k, :], mk)


def _values_kernel(x_ref, y_ref, t_ref, p_ref, b_ref, tm_ref, P_ref,
                   v0, v1, v2, v3, v4, v5, v6, v7, v8, base_ref):
    outs = (v0, v1, v2, v3, v4, v5, v6, v7, v8)
    t = t_ref[...]
    b = b_ref[...]
    tm0 = jnp.max(tm_ref[0, :])
    tm1 = jnp.max(tm_ref[1, :])
    tm2 = jnp.max(tm_ref[2, :])
    tm3 = jnp.max(tm_ref[3, :])
    tmsel = jnp.where(b == 0.0, tm0,
                      jnp.where(b == 1.0, tm1,
                                jnp.where(b == 2.0, tm2, tm3)))
    tn = t / tmsel
    p2 = (p_ref[...] + 1.0) * 0.5
    g = b * 2.0 + p2
    pix = x_ref[...] + float(W) * y_ref[...]
    base_ref[...] = (g * float(HW) + pix).astype(jnp.int32)

    w1c = P_ref[0:16, 0:1]
    b1c = P_ref[0:16, 1:2]
    w2m = P_ref[0:16, 2:18]
    b2c = P_ref[0:16, 18:19]
    w3c = P_ref[0:16, 19:20]
    b3s = P_ref[0, 20]

    tn1 = tn.reshape(1, tn.shape[0])
    dn = (((0,), (0,)), ((), ()))
    for c in range(C):
        u1 = tn1 - float(c) / float(C - 1)
        h1 = _leaky(w1c * u1 + b1c)
        h2 = lax.dot_general(w2m, h1, dn,
                             preferred_element_type=jnp.float32)
        h2 = _leaky(h2 + b2c)
        v = lax.dot_general(w3c, h2, dn,
                            preferred_element_type=jnp.float32)
        val = tn1 * (v + b3s)
        outs[c][...] = val.reshape(val.shape[1])


def _sc_scatter_kernel(v0, v1, v2, v3, v4, v5, v6, v7, v8, base_hbm,
                       out_hbm, base_v, idx_v, val_v, zbuf, acc):
    cid = lax.axis_index("c")
    sid = lax.axis_index("s")
    vrefs = (v0, v1, v2, v3, v4, v5, v6, v7, v8)
    e0 = sid * _SC_EV

    def _zero_body(i, carry):
        zbuf[pl.ds(i * 16, 16)] = jnp.zeros((16,), jnp.float32)
        return carry

    def _zero_acc(nplanes):
        lax.fori_loop(0, _ZB // 16, _zero_body, 0)
        for q in range(2 * nplanes):
            pltpu.sync_copy(
                zbuf, acc.at[pl.ds(sid * nplanes * 21600 + q * _ZB, _ZB)])

    def _scatter(first_bin, nb):
        def body(j, carry):
            off = e0 + j * _SC_CHUNK
            pltpu.sync_copy(base_hbm.at[pl.ds(off, _SC_CHUNK)], base_v)
            for dc in range(nb):
                if dc > 0:
                    def shift(i, carry2):
                        idx_v[pl.ds(i * 16, 16)] = (
                            base_v[pl.ds(i * 16, 16)] + dc * GHW)
                        return carry2

                    lax.fori_loop(0, _SC_CHUNK // 16, shift, 0)
                pltpu.sync_copy(vrefs[first_bin + dc].at[pl.ds(off, _SC_CHUNK)],
                                val_v)
                if dc > 0:
                    pltpu.sync_copy(val_v, acc.at[idx_v], add=True)
                else:
                    pltpu.sync_copy(val_v, acc.at[base_v], add=True)
            return carry

        lax.fori_loop(0, _SC_EV // _SC_CHUNK, body, 0)

    def _drain(first_bin, nb):
        nch = nb * 8 * (HW // _ZB) // 16

        def body(m, carry):
            cidx = sid * nch + m
            p_id = cidx // (HW // _ZB)
            w = (cidx % (HW // _ZB)) * _ZB
            dc = p_id // 8
            g = p_id % 8
            out_off = (g * C + first_bin + dc) * HW + w
            pltpu.sync_copy(acc.at[pl.ds(cidx * _ZB, _ZB)], zbuf)
            pltpu.sync_copy(zbuf, out_hbm.at[pl.ds(out_off, _ZB)])
            return carry

        lax.fori_loop(0, nch, body, 0)

    _zero_acc(4)
    plsc.subcore_barrier()

    @pl.when(cid == 0)
    def _():
        _scatter(0, 4)

    @pl.when(cid == 1)
    def _():
        _scatter(4, 4)

    plsc.subcore_barrier()

    @pl.when(cid == 0)
    def _():
        _drain(0, 4)

    @pl.when(cid == 1)
    def _():
        _drain(4, 4)

    plsc.subcore_barrier()

    _zero_acc(1)
    plsc.subcore_barrier()

    @pl.when(cid == 0)
    def _():
        _scatter(8, 1)

    plsc.subcore_barrier()

    @pl.when(cid == 0)
    def _():
        _drain(8, 1)


def kernel(events, w1, b1, w2, b2, w3, b3):
    ev = jnp.pad(events.astype(jnp.float32), ((0, N_PAD - N), (0, 0)))
    x = ev[:, 0]
    y = ev[:, 1]
    t = ev[:, 2]
    p = ev[:, 3]
    b = ev[:, 4]

    t2d = t.reshape(N_PAD // 128, 128)
    b2d = b.reshape(N_PAD // 128, 128)
    grid_a = N_PAD // _A_CHUNK
    tmax8 = pl.pallas_call(
        _tmax_kernel,
        grid=(grid_a,),
        in_specs=[
            pl.BlockSpec((_A_CHUNK // 128, 128), lambda i: (i, 0)),
            pl.BlockSpec((_A_CHUNK // 128, 128), lambda i: (i, 0)),
        ],
        out_specs=pl.BlockSpec((8, 128), lambda i: (0, 0)),
        out_shape=jax.ShapeDtypeStruct((8, 128), jnp.float32),
    )(t2d, b2d)

    P = jnp.zeros((16, 128), jnp.float32)
    P = P.at[:, 0].set(jnp.broadcast_to(w1[0], (16,)))
    P = P.at[:, 1].set(b1)
    P = P.at[:, 2:18].set(w2)
    P = P.at[:, 18].set(b2)
    P = P.at[:, 19].set(w3[:, 0])
    P = P.at[0, 20].set(b3[0])

    grid_b = N_PAD // _B_CHUNK
    ev_spec = pl.BlockSpec((_B_CHUNK,), lambda i: (i,))
    out_1d = jax.ShapeDtypeStruct((N_PAD,), jnp.float32)
    outs = pl.pallas_call(
        _values_kernel,
        grid=(grid_b,),
        in_specs=[ev_spec, ev_spec, ev_spec, ev_spec, ev_spec,
                  pl.BlockSpec((8, 128), lambda i: (0, 0)),
                  pl.BlockSpec((16, 128), lambda i: (0, 0))],
        out_specs=[ev_spec] * 9 + [ev_spec],
        out_shape=[out_1d] * 9 + [jax.ShapeDtypeStruct((N_PAD,), jnp.int32)],
    )(x, y, t, p, b, tmax8, P)
    vals = outs[:9]
    base = outs[9]

    mesh = plsc.VectorSubcoreMesh(core_axis_name="c", subcore_axis_name="s")
    sc = functools.partial(
        pl.kernel,
        mesh=mesh,
        out_type=jax.ShapeDtypeStruct((2 * C * HW * B,), jnp.float32),
        scratch_types=[
            pltpu.VMEM((_SC_CHUNK,), jnp.int32),
            pltpu.VMEM((_SC_CHUNK,), jnp.int32),
            pltpu.VMEM((_SC_CHUNK,), jnp.float32),
            pltpu.VMEM((_ZB,), jnp.float32),
            pltpu.VMEM_SHARED((4 * GHW,), jnp.float32),
        ],
    )(_sc_scatter_kernel)
    out_flat = sc(*vals, base)
    return out_flat.reshape(B, 2 * C, H, W)


# speedup vs baseline: 21.9616x; 1.0288x over previous
import functools

import jax
import jax.numpy as jnp
from jax import lax
from jax.experimental import pallas as pl
from jax.experimental.pallas import tpu as pltpu
from jax.experimental.pallas import tpu_sc as plsc

C = 9
H = 180
W = 240
B = 4
N = 1000000
HW = H * W
GHW = 8 * HW
N_PAD = 1 << 20
NB0 = 5
NB1 = 4

_A_CHUNK = 131072
_B_CHUNK = 16384
_SC_EV = N_PAD // 16
_SC_CHUNK = 4096
_ZB = 10800


def _leaky(v):
    return jnp.where(v >= 0, v, 0.1 * v)


def _tmax_kernel(t_ref, b_ref, o_ref):
    pid = pl.program_id(0)

    @pl.when(pid == 0)
    def _():
        o_ref[...] = jnp.zeros_like(o_ref)

    t = t_ref[...]
    b = b_ref[...]
    for k in range(B):
        mk = jnp.max(jnp.where(b == float(k), t, 0.0), axis=0)
        o_ref[k, :] = jnp.maximum(o_ref[k, :], mk)


def _values_kernel(x_ref, y_ref, t_ref, p_ref, b_ref, tm_ref, P_ref,
                   v0, v1, v2, v3, v4, v5, v6, v7, v8, base_ref):
    outs = (v0, v1, v2, v3, v4, v5, v6, v7, v8)
    t = t_ref[...]
    b = b_ref[...]
    tm0 = jnp.max(tm_ref[0, :])
    tm1 = jnp.max(tm_ref[1, :])
    tm2 = jnp.max(tm_ref[2, :])
    tm3 = jnp.max(tm_ref[3, :])
    tmsel = jnp.where(b == 0.0, tm0,
                      jnp.where(b == 1.0, tm1,
                                jnp.where(b == 2.0, tm2, tm3)))
    tn = t / tmsel
    p2 = (p_ref[...] + 1.0) * 0.5
    g = b * 2.0 + p2
    pix = x_ref[...] + float(W) * y_ref[...]
    base_ref[...] = (g * float(HW) + pix).astype(jnp.int32)

    w1c = P_ref[0:16, 0:1]
    b1c = P_ref[0:16, 1:2]
    w2m = P_ref[0:16, 2:18]
    b2c = P_ref[0:16, 18:19]
    w3c = P_ref[0:16, 19:20]
    b3s = P_ref[0, 20]

    tn1 = tn.reshape(1, tn.shape[0])
    dn = (((0,), (0,)), ((), ()))
    for c in range(C):
        u1 = tn1 - float(c) / float(C - 1)
        h1 = _leaky(w1c * u1 + b1c)
        h2 = lax.dot_general(w2m, h1, dn,
                             preferred_element_type=jnp.float32)
        h2 = _leaky(h2 + b2c)
        v = lax.dot_general(w3c, h2, dn,
                            preferred_element_type=jnp.float32)
        val = tn1 * (v + b3s)
        outs[c][...] = val.reshape(val.shape[1])


def _sc_scatter_kernel(v0, v1, v2, v3, v4, v5, v6, v7, v8, base_hbm,
                       out_hbm, base_v0, base_v1, idx_v, val_v0, val_v1,
                       zbuf, acc, semb0, semb1, semv0, semv1):
    cid = lax.axis_index("c")
    sid = lax.axis_index("s")
    vrefs = (v0, v1, v2, v3, v4, v5, v6, v7, v8)
    e0 = sid * _SC_EV

    def _zero_body(i, carry):
        zbuf[pl.ds(i * 16, 16)] = jnp.zeros((16,), jnp.float32)
        return carry

    def _zero_acc(nplanes):
        lax.fori_loop(0, _ZB // 16, _zero_body, 0)
        for q in range(2 * nplanes):
            pltpu.sync_copy(
                zbuf, acc.at[pl.ds(sid * nplanes * 21600 + q * _ZB, _ZB)])

    bases = (base_v0, base_v1)
    vbufs = (val_v0, val_v1)
    sembs = (semb0, semb1)
    semvs = (semv0, semv1)

    def _scatter(first_bin, nb):
        nchunks = _SC_EV // _SC_CHUNK
        nq = nb * nchunks
        hb = {}
        hv = {}

        def start_base(j):
            if j < nchunks:
                hb[j] = pltpu.async_copy(
                    base_hbm.at[pl.ds(e0 + j * _SC_CHUNK, _SC_CHUNK)],
                    bases[j % 2], sembs[j % 2])

        def start_val(q):
            if q < nq:
                j, dc = divmod(q, nb)
                hv[q] = pltpu.async_copy(
                    vrefs[first_bin + dc].at[
                        pl.ds(e0 + j * _SC_CHUNK, _SC_CHUNK)],
                    vbufs[q % 2], semvs[q % 2])

        start_base(0)
        start_val(0)
        for j in range(nchunks):
            hb[j].wait()
            start_base(j + 1)
            for dc in range(nb):
                q = j * nb + dc
                hv[q].wait()
                start_val(q + 1)
                if dc > 0:
                    def shift(i, carry2):
                        idx_v[pl.ds(i * 16, 16)] = (
                            bases[j % 2][pl.ds(i * 16, 16)] + dc * GHW)
                        return carry2

                    lax.fori_loop(0, _SC_CHUNK // 16, shift, 0)
                    idxref = idx_v
                else:
                    idxref = bases[j % 2]
                pltpu.sync_copy(vbufs[q % 2], acc.at[idxref], add=True)

    def _drain(first_bin, nb):
        nch = nb * 8 * (HW // _ZB) // 16

        def body(m, carry):
            cidx = sid * nch + m
            p_id = cidx // (HW // _ZB)
            w = (cidx % (HW // _ZB)) * _ZB
            dc = p_id // 8
            g = p_id % 8
            out_off = (g * C + first_bin + dc) * HW + w
            pltpu.sync_copy(acc.at[pl.ds(cidx * _ZB, _ZB)], zbuf)
            pltpu.sync_copy(zbuf, out_hbm.at[pl.ds(out_off, _ZB)])
            return carry

        lax.fori_loop(0, nch, body, 0)

    _zero_acc(4)
    plsc.subcore_barrier()

    @pl.when(cid == 0)
    def _():
        _scatter(0, 4)

    @pl.when(cid == 1)
    def _():
        _scatter(4, 4)

    plsc.subcore_barrier()

    @pl.when(cid == 0)
    def _():
        _drain(0, 4)

    @pl.when(cid == 1)
    def _():
        _drain(4, 4)

    plsc.subcore_barrier()

    _zero_acc(1)
    plsc.subcore_barrier()

    @pl.when(cid == 0)
    def _():
        _scatter(8, 1)

    plsc.subcore_barrier()

    @pl.when(cid == 0)
    def _():
        _drain(8, 1)


def kernel(events, w1, b1, w2, b2, w3, b3):
    ev = jnp.pad(events.astype(jnp.float32), ((0, N_PAD - N), (0, 0)))
    x = ev[:, 0]
    y = ev[:, 1]
    t = ev[:, 2]
    p = ev[:, 3]
    b = ev[:, 4]

    t2d = t.reshape(N_PAD // 128, 128)
    b2d = b.reshape(N_PAD // 128, 128)
    grid_a = N_PAD // _A_CHUNK
    tmax8 = pl.pallas_call(
        _tmax_kernel,
        grid=(grid_a,),
        in_specs=[
            pl.BlockSpec((_A_CHUNK // 128, 128), lambda i: (i, 0)),
            pl.BlockSpec((_A_CHUNK // 128, 128), lambda i: (i, 0)),
        ],
        out_specs=pl.BlockSpec((8, 128), lambda i: (0, 0)),
        out_shape=jax.ShapeDtypeStruct((8, 128), jnp.float32),
    )(t2d, b2d)

    P = jnp.zeros((16, 128), jnp.float32)
    P = P.at[:, 0].set(jnp.broadcast_to(w1[0], (16,)))
    P = P.at[:, 1].set(b1)
    P = P.at[:, 2:18].set(w2)
    P = P.at[:, 18].set(b2)
    P = P.at[:, 19].set(w3[:, 0])
    P = P.at[0, 20].set(b3[0])

    grid_b = N_PAD // _B_CHUNK
    ev_spec = pl.BlockSpec((_B_CHUNK,), lambda i: (i,))
    out_1d = jax.ShapeDtypeStruct((N_PAD,), jnp.float32)
    outs = pl.pallas_call(
        _values_kernel,
        grid=(grid_b,),
        in_specs=[ev_spec, ev_spec, ev_spec, ev_spec, ev_spec,
                  pl.BlockSpec((8, 128), lambda i: (0, 0)),
                  pl.BlockSpec((16, 128), lambda i: (0, 0))],
        out_specs=[ev_spec] * 9 + [ev_spec],
        out_shape=[out_1d] * 9 + [jax.ShapeDtypeStruct((N_PAD,), jnp.int32)],
    )(x, y, t, p, b, tmax8, P)
    vals = outs[:9]
    base = outs[9]

    mesh = plsc.VectorSubcoreMesh(core_axis_name="c", subcore_axis_name="s")
    sc = functools.partial(
        pl.kernel,
        mesh=mesh,
        out_type=jax.ShapeDtypeStruct((2 * C * HW * B,), jnp.float32),
        scratch_types=[
            pltpu.VMEM((_SC_CHUNK,), jnp.int32),
            pltpu.VMEM((_SC_CHUNK,), jnp.int32),
            pltpu.VMEM((_SC_CHUNK,), jnp.int32),
            pltpu.VMEM((_SC_CHUNK,), jnp.float32),
            pltpu.VMEM((_SC_CHUNK,), jnp.float32),
            pltpu.VMEM((_ZB,), jnp.float32),
            pltpu.VMEM_SHARED((4 * GHW,), jnp.float32),
            pltpu.SemaphoreType.DMA,
            pltpu.SemaphoreType.DMA,
            pltpu.SemaphoreType.DMA,
            pltpu.SemaphoreType.DMA,
        ],
    )(_sc_scatter_kernel)
    out_flat = sc(*vals, base)
    return out_flat.reshape(B, 2 * C, H, W)


# speedup vs baseline: 22.0586x; 1.0044x over previous
import functools

import jax
import jax.numpy as jnp
from jax import lax
from jax.experimental import pallas as pl
from jax.experimental.pallas import tpu as pltpu
from jax.experimental.pallas import tpu_sc as plsc

C = 9
H = 180
W = 240
B = 4
N = 1000000
HW = H * W
GHW = 8 * HW
N_PAD = 1 << 20
NB0 = 5
NB1 = 4

_A_CHUNK = 131072
_B_CHUNK = 16384
_SC_EV = N_PAD // 16
_SC_CHUNK = 4096
_ZB = 10800


def _leaky(v):
    return jnp.where(v >= 0, v, 0.1 * v)


def _tmax_kernel(t_ref, b_ref, o_ref):
    pid = pl.program_id(0)

    @pl.when(pid == 0)
    def _():
        o_ref[...] = jnp.zeros_like(o_ref)

    t = t_ref[...]
    b = b_ref[...]
    for k in range(B):
        mk = jnp.max(jnp.where(b == float(k), t, 0.0), axis=0)
        o_ref[k, :] = jnp.maximum(o_ref[k, :], mk)


def _values_kernel(x_ref, y_ref, t_ref, p_ref, b_ref, tm_ref, P_ref,
                   v0, v1, v2, v3, v4, v5, v6, v7, v8, base_ref):
    outs = (v0, v1, v2, v3, v4, v5, v6, v7, v8)
    t = t_ref[...]
    b = b_ref[...]
    tm0 = jnp.max(tm_ref[0, :])
    tm1 = jnp.max(tm_ref[1, :])
    tm2 = jnp.max(tm_ref[2, :])
    tm3 = jnp.max(tm_ref[3, :])
    tmsel = jnp.where(b == 0.0, tm0,
                      jnp.where(b == 1.0, tm1,
                                jnp.where(b == 2.0, tm2, tm3)))
    tn = t / tmsel
    p2 = (p_ref[...] + 1.0) * 0.5
    g = b * 2.0 + p2
    pix = x_ref[...] + float(W) * y_ref[...]
    base_ref[...] = (g * float(HW) + pix).astype(jnp.int32)

    w1c = P_ref[0:16, 0:1]
    b1c = P_ref[0:16, 1:2]
    w2m = P_ref[0:16, 2:18]
    b2c = P_ref[0:16, 18:19]
    w3c = P_ref[0:16, 19:20]
    b3s = P_ref[0, 20]

    tn1 = tn.reshape(1, tn.shape[0])
    dn = (((0,), (0,)), ((), ()))
    for c in range(C):
        u1 = tn1 - float(c) / float(C - 1)
        h1 = _leaky(w1c * u1 + b1c)
        h2 = lax.dot_general(w2m, h1, dn,
                             preferred_element_type=jnp.float32)
        h2 = _leaky(h2 + b2c)
        v = lax.dot_general(w3c, h2, dn,
                            preferred_element_type=jnp.float32)
        val = tn1 * (v + b3s)
        outs[c][...] = val.reshape(val.shape[1])


def _sc_scatter_kernel(v0, v1, v2, v3, v4, v5, v6, v7, v8, base_hbm,
                       out_hbm, base_v0, base_v1, idx_v, val_v0, val_v1,
                       zbuf, acc, semb0, semb1, semv0, semv1):
    cid = lax.axis_index("c")
    sid = lax.axis_index("s")
    vrefs = (v0, v1, v2, v3, v4, v5, v6, v7, v8)
    e0 = sid * _SC_EV

    def _zero_body(i, carry):
        zbuf[pl.ds(i * 16, 16)] = jnp.zeros((16,), jnp.float32)
        return carry

    def _zero_acc(nplanes):
        lax.fori_loop(0, _ZB // 16, _zero_body, 0)
        for q in range(2 * nplanes):
            pltpu.sync_copy(
                zbuf, acc.at[pl.ds(sid * nplanes * 21600 + q * _ZB, _ZB)])

    bases = (base_v0, base_v1)
    vbufs = (val_v0, val_v1)
    sembs = (semb0, semb1)
    semvs = (semv0, semv1)

    def _scatter(first_bin, nb):
        nchunks = _SC_EV // _SC_CHUNK
        nq = nb * nchunks
        hb = {}
        hv = {}

        def start_base(j):
            if j < nchunks:
                hb[j] = pltpu.async_copy(
                    base_hbm.at[pl.ds(e0 + j * _SC_CHUNK, _SC_CHUNK)],
                    bases[j % 2], sembs[j % 2])

        def start_val(q):
            if q < nq:
                j, dc = divmod(q, nb)
                hv[q] = pltpu.async_copy(
                    vrefs[first_bin + dc].at[
                        pl.ds(e0 + j * _SC_CHUNK, _SC_CHUNK)],
                    vbufs[q % 2], semvs[q % 2])

        start_base(0)
        start_val(0)
        for j in range(nchunks):
            hb[j].wait()
            start_base(j + 1)
            for dc in range(nb):
                q = j * nb + dc
                hv[q].wait()
                start_val(q + 1)
                if dc > 0:
                    def shift(i, carry2):
                        idx_v[pl.ds(i * 16, 16)] = (
                            bases[j % 2][pl.ds(i * 16, 16)] + dc * GHW)
                        return carry2

                    lax.fori_loop(0, _SC_CHUNK // 16, shift, 0)
                    idxref = idx_v
                else:
                    idxref = bases[j % 2]
                pltpu.sync_copy(vbufs[q % 2], acc.at[idxref], add=True)

    def _drain(first_bin, nb):
        nch = nb * 8 * (HW // _ZB) // 16

        def body(m, carry):
            cidx = sid * nch + m
            p_id = cidx // (HW // _ZB)
            w = (cidx % (HW // _ZB)) * _ZB
            dc = p_id // 8
            g = p_id % 8
            out_off = (g * C + first_bin + dc) * HW + w
            pltpu.sync_copy(acc.at[pl.ds(cidx * _ZB, _ZB)], zbuf)
            pltpu.sync_copy(zbuf, out_hbm.at[pl.ds(out_off, _ZB)])
            return carry

        lax.fori_loop(0, nch, body, 0)

    _zero_acc(4)
    plsc.subcore_barrier()

    @pl.when(cid == 0)
    def _():
        _scatter(0, 4)

    @pl.when(cid == 1)
    def _():
        _scatter(4, 4)

    plsc.subcore_barrier()

    @pl.when(cid == 0)
    def _():
        _drain(0, 4)

    @pl.when(cid == 1)
    def _():
        _drain(4, 4)

    plsc.subcore_barrier()

    _zero_acc(1)
    plsc.subcore_barrier()

    @pl.when(cid == 0)
    def _():
        _scatter(8, 1)

    plsc.subcore_barrier()

    @pl.when(cid == 0)
    def _():
        _drain(8, 1)


def kernel(events, w1, b1, w2, b2, w3, b3):
    ev = jnp.pad(events.astype(jnp.float32), ((0, N_PAD - N), (0, 0)))
    x = ev[:, 0]
    y = ev[:, 1]
    t = ev[:, 2]
    p = ev[:, 3]
    b = ev[:, 4]

    t2d = t.reshape(N_PAD // 128, 128)
    b2d = b.reshape(N_PAD // 128, 128)
    grid_a = N_PAD // _A_CHUNK
    tmax8 = pl.pallas_call(
        _tmax_kernel,
        grid=(grid_a,),
        in_specs=[
            pl.BlockSpec((_A_CHUNK // 128, 128), lambda i: (i, 0)),
            pl.BlockSpec((_A_CHUNK // 128, 128), lambda i: (i, 0)),
        ],
        out_specs=pl.BlockSpec((8, 128), lambda i: (0, 0)),
        out_shape=jax.ShapeDtypeStruct((8, 128), jnp.float32),
    )(t2d, b2d)

    P = jnp.zeros((16, 128), jnp.float32)
    P = P.at[:, 0].set(jnp.broadcast_to(w1[0], (16,)))
    P = P.at[:, 1].set(b1)
    P = P.at[:, 2:18].set(w2)
    P = P.at[:, 18].set(b2)
    P = P.at[:, 19].set(w3[:, 0])
    P = P.at[0, 20].set(b3[0])

    grid_b = N_PAD // _B_CHUNK
    ev_spec = pl.BlockSpec((_B_CHUNK,), lambda i: (i,))
    out_1d = jax.ShapeDtypeStruct((N_PAD,), jnp.float32)
    outs = pl.pallas_call(
        _values_kernel,
        grid=(grid_b,),
        in_specs=[ev_spec, ev_spec, ev_spec, ev_spec, ev_spec,
                  pl.BlockSpec((8, 128), lambda i: (0, 0)),
                  pl.BlockSpec((16, 128), lambda i: (0, 0))],
        out_specs=[ev_spec] * 9 + [ev_spec],
        out_shape=[out_1d] * 9 + [jax.ShapeDtypeStruct((N_PAD,), jnp.int32)],
    )(x, y, t, p, b, tmax8, P)
    vals = outs[:9]
    base = outs[9]

    mesh = plsc.VectorSubcoreMesh(core_axis_name="c", subcore_axis_name="s",
                                  num_cores=2)
    sc = functools.partial(
        pl.kernel,
        mesh=mesh,
        out_type=jax.ShapeDtypeStruct((2 * C * HW * B,), jnp.float32),
        scratch_types=[
            pltpu.VMEM((_SC_CHUNK,), jnp.int32),
            pltpu.VMEM((_SC_CHUNK,), jnp.int32),
            pltpu.VMEM((_SC_CHUNK,), jnp.int32),
            pltpu.VMEM((_SC_CHUNK,), jnp.float32),
            pltpu.VMEM((_SC_CHUNK,), jnp.float32),
            pltpu.VMEM((_ZB,), jnp.float32),
            pltpu.VMEM_SHARED((4 * GHW,), jnp.float32),
            pltpu.SemaphoreType.DMA,
            pltpu.SemaphoreType.DMA,
            pltpu.SemaphoreType.DMA,
            pltpu.SemaphoreType.DMA,
        ],
    )(_sc_scatter_kernel)
    out_flat = sc(*vals, base)
    return out_flat.reshape(B, 2 * C, H, W)
